# Initial kernel scaffold; baseline (speedup 1.0000x reference)
import jax
import jax.numpy as jnp
from jax.experimental import pallas as pl


def kernel(inputs, table):
    raise NotImplementedError("write your pallas kernel here")


# speedup vs baseline: 3.8464x; 3.8464x over previous
import functools

import jax
import jax.numpy as jnp
import numpy as np
from jax import lax
from jax.experimental import pallas as pl
from jax.experimental.pallas import tpu as pltpu
from jax.experimental.pallas import tpu_sc as plsc

VOCAB = 100000
D_MODEL = 128
BATCH = 4096
SEQ = 200
SCALE = float(np.sqrt(D_MODEL))

IDX_CHUNKS = 2
CHUNK = SEQ // IDX_CHUNKS


def _positional_encoding(length, depth):
    positions = np.arange(length)[:, np.newaxis]
    depths = np.arange(depth // 2)[np.newaxis, :] / (depth // 2)
    angle_rates = 1.0 / (10000 ** depths)
    angle_rads = positions * angle_rates
    return np.concatenate(
        [np.sin(angle_rads), np.cos(angle_rads)], axis=-1
    ).astype(np.float32)


_PE = _positional_encoding(SEQ, D_MODEL)


def _make_sc_kernel():
    info = plsc.get_sparse_core_info()
    nc, ns, lanes = info.num_cores, info.num_subcores, info.num_lanes
    nw = nc * ns
    seq_per_w = BATCH // nw
    mesh = plsc.VectorSubcoreMesh(core_axis_name="c", subcore_axis_name="s")

    @functools.partial(
        pl.kernel,
        mesh=mesh,
        out_type=jax.ShapeDtypeStruct((BATCH, SEQ, D_MODEL), jnp.float32),
        scratch_types=[
            pltpu.VMEM((IDX_CHUNKS, CHUNK), jnp.int32),
            pltpu.VMEM((SEQ, D_MODEL), jnp.float32),
            pltpu.VMEM((SEQ, D_MODEL), jnp.float32),
            pltpu.SemaphoreType.DMA,
        ],
    )
    def k(idx_hbm, table_hbm, pe_hbm, out_hbm, idx_v, rows_v, pe_v, sem):
        wid = lax.axis_index("s") * nc + lax.axis_index("c")
        base = wid * seq_per_w
        pltpu.sync_copy(pe_hbm, pe_v)

        def seq_body(i, carry):
            seq = base + i
            pltpu.sync_copy(idx_hbm.at[seq], idx_v)
            for c in range(IDX_CHUNKS):
                pltpu.async_copy(
                    table_hbm.at[idx_v.at[c]],
                    rows_v.at[pl.ds(c * CHUNK, CHUNK)],
                    sem,
                ).wait()

            def row_body(j, carry2):
                for v in range(D_MODEL // lanes):
                    sl = pl.ds(v * lanes, lanes)
                    rows_v[j, sl] = rows_v[j, sl] * SCALE + pe_v[j, sl]
                return carry2

            lax.fori_loop(0, SEQ, row_body, 0)
            pltpu.sync_copy(rows_v, out_hbm.at[seq])
            return carry

        lax.fori_loop(0, seq_per_w, seq_body, 0)

    return k


_sc_kernel = _make_sc_kernel()


def kernel(inputs, table):
    idx = inputs.reshape(BATCH, IDX_CHUNKS, CHUNK)
    return _sc_kernel(idx, table, jnp.asarray(_PE))


# speedup vs baseline: 3.9648x; 1.0308x over previous
import functools

import jax
import jax.numpy as jnp
import numpy as np
from jax import lax
from jax.experimental import pallas as pl
from jax.experimental.pallas import tpu as pltpu
from jax.experimental.pallas import tpu_sc as plsc

VOCAB = 100000
D_MODEL = 128
BATCH = 4096
SEQ = 200
SCALE = float(np.sqrt(D_MODEL))

CHUNK = 100
UNITS = BATCH * SEQ // CHUNK


def _positional_encoding(length, depth):
    positions = np.arange(length)[:, np.newaxis]
    depths = np.arange(depth // 2)[np.newaxis, :] / (depth // 2)
    angle_rates = 1.0 / (10000 ** depths)
    angle_rads = positions * angle_rates
    return np.concatenate(
        [np.sin(angle_rads), np.cos(angle_rads)], axis=-1
    ).astype(np.float32)


_PE = _positional_encoding(SEQ, D_MODEL)


def _make_sc_kernel():
    info = plsc.get_sparse_core_info()
    nc, ns, lanes = info.num_cores, info.num_subcores, info.num_lanes
    nw = nc * ns
    u_per_w = UNITS // nw
    iters = u_per_w // 2
    nvec = D_MODEL // lanes
    mesh = plsc.VectorSubcoreMesh(core_axis_name="c", subcore_axis_name="s")

    @functools.partial(
        pl.kernel,
        mesh=mesh,
        out_type=jax.ShapeDtypeStruct((UNITS, CHUNK, D_MODEL), jnp.float32),
        scratch_types=[
            pltpu.VMEM((u_per_w, CHUNK), jnp.int32),
            pltpu.VMEM((CHUNK, D_MODEL), jnp.float32),
            pltpu.VMEM((CHUNK, D_MODEL), jnp.float32),
            pltpu.VMEM((CHUNK, D_MODEL), jnp.float32),
            pltpu.VMEM((CHUNK, D_MODEL), jnp.float32),
            pltpu.VMEM((SEQ, D_MODEL), jnp.float32),
            pltpu.SemaphoreType.DMA,
            pltpu.SemaphoreType.DMA,
            pltpu.SemaphoreType.DMA,
            pltpu.SemaphoreType.DMA,
        ],
    )
    def k(idx_hbm, table_hbm, pe_hbm, out_hbm,
          idx_v, in0, in1, out0, out1, pe_v, g0, g1, s0, s1):
        rows_in = (in0, in1)
        rows_out = (out0, out1)
        gsem = (g0, g1)
        ssem = (s0, s1)
        wid = lax.axis_index("s") * nc + lax.axis_index("c")
        ubase = wid * u_per_w
        pltpu.sync_copy(pe_hbm, pe_v)
        pltpu.sync_copy(idx_hbm.at[pl.ds(ubase, u_per_w)], idx_v)
        for b in range(2):
            pltpu.async_copy(table_hbm.at[idx_v.at[b]], rows_in[b], gsem[b])

        def body(i, carry):
            for b in range(2):
                @pl.when(i > 0)
                def _wait_store(b=b):
                    pltpu.make_async_copy(
                        rows_out[b], out_hbm.at[ubase], ssem[b]
                    ).wait()
            for b in range(2):
                u = ubase + 2 * i + b
                pltpu.make_async_copy(
                    out_hbm.at[ubase], rows_in[b], gsem[b]
                ).wait()

                def rbody(j, c2, b=b):
                    for r in range(2):
                        jj = 2 * j + r
                        for v in range(nvec):
                            sl = pl.ds(v * lanes, lanes)
                            rows_out[b][jj, sl] = (
                                rows_in[b][jj, sl] * SCALE
                                + pe_v[b * CHUNK + jj, sl]
                            )
                    return c2

                lax.fori_loop(0, CHUNK // 2, rbody, 0)
                pltpu.async_copy(rows_out[b], out_hbm.at[u], ssem[b])

                @pl.when(i < iters - 1)
                def _fire_next(i=i, b=b):
                    pltpu.async_copy(
                        table_hbm.at[idx_v.at[2 * (i + 1) + b]],
                        rows_in[b],
                        gsem[b],
                    )
            return carry

        lax.fori_loop(0, iters, body, 0)
        for b in range(2):
            pltpu.make_async_copy(
                rows_out[b], out_hbm.at[ubase], ssem[b]
            ).wait()

    return k


_sc_kernel = _make_sc_kernel()


def kernel(inputs, table):
    idx = inputs.reshape(UNITS, CHUNK)
    out = _sc_kernel(idx, table, jnp.asarray(_PE))
    return out.reshape(BATCH, SEQ, D_MODEL)


# speedup vs baseline: 6.3577x; 1.6035x over previous
import functools

import jax
import jax.numpy as jnp
import numpy as np
from jax import lax
from jax.experimental import pallas as pl
from jax.experimental.pallas import tpu as pltpu
from jax.experimental.pallas import tpu_sc as plsc

VOCAB = 100000
D_MODEL = 128
BATCH = 4096
SEQ = 200
SCALE = float(np.sqrt(D_MODEL))

CHUNK = 40
NCHUNK = SEQ // CHUNK
UNITS = BATCH * NCHUNK


def _positional_encoding(length, depth):
    positions = np.arange(length)[:, np.newaxis]
    depths = np.arange(depth // 2)[np.newaxis, :] / (depth // 2)
    angle_rates = 1.0 / (10000 ** depths)
    angle_rads = positions * angle_rates
    return np.concatenate(
        [np.sin(angle_rads), np.cos(angle_rads)], axis=-1
    ).astype(np.float32)


_PE = _positional_encoding(SEQ, D_MODEL)


def _make_sc_kernel():
    info = plsc.get_sparse_core_info()
    nc, ns, lanes = info.num_cores, info.num_subcores, info.num_lanes
    nw = nc * ns
    seq_per_w = BATCH // nw
    u_per_w = UNITS // nw
    nvec = D_MODEL // lanes
    mesh = plsc.VectorSubcoreMesh(core_axis_name="c", subcore_axis_name="s")

    @functools.partial(
        pl.kernel,
        mesh=mesh,
        out_type=jax.ShapeDtypeStruct((BATCH, SEQ, D_MODEL), jnp.float32),
        scratch_types=[
            pltpu.VMEM((u_per_w // 2, CHUNK), jnp.int32),
            pltpu.VMEM((SEQ, D_MODEL), jnp.float32),
        ]
        + [pltpu.VMEM((CHUNK, D_MODEL), jnp.float32)] * (2 * NCHUNK)
        + [pltpu.SemaphoreType.DMA] * (2 * NCHUNK),
    )
    def k(idx_hbm, table_hbm, pe_hbm, out_hbm, idx_v, pe_v, *bufs):
        rows_in = bufs[:NCHUNK]
        rows_out = bufs[NCHUNK:2 * NCHUNK]
        gsem = bufs[2 * NCHUNK:3 * NCHUNK]
        ssem = bufs[3 * NCHUNK:4 * NCHUNK]
        wid = lax.axis_index("s") * nc + lax.axis_index("c")
        sbase = wid * seq_per_w
        half_u = u_per_w // 2
        half_i = seq_per_w // 2
        pltpu.sync_copy(pe_hbm, pe_v)
        pltpu.sync_copy(idx_hbm.at[pl.ds(wid * u_per_w, half_u)], idx_v)
        for c in range(NCHUNK):
            pltpu.async_copy(table_hbm.at[idx_v.at[c]], rows_in[c], gsem[c])

        def body(i, carry):
            seq = sbase + i
            for c in range(NCHUNK):
                pltpu.make_async_copy(
                    out_hbm.at[sbase, pl.ds(c * CHUNK, CHUNK)],
                    rows_in[c], gsem[c],
                ).wait()

            @pl.when(i == half_i - 1)
            def _refill_idx():
                pltpu.sync_copy(
                    idx_hbm.at[pl.ds(wid * u_per_w + half_u, half_u)], idx_v
                )

            for c in range(NCHUNK):
                @pl.when(i > 0)
                def _wait_store(c=c):
                    pltpu.make_async_copy(
                        rows_out[c],
                        out_hbm.at[sbase, pl.ds(c * CHUNK, CHUNK)],
                        ssem[c],
                    ).wait()

                def rbody(j, c2, c=c):
                    for r in range(4):
                        jj = 4 * j + r
                        for v in range(nvec):
                            sl = pl.ds(v * lanes, lanes)
                            rows_out[c][jj, sl] = (
                                rows_in[c][jj, sl] * SCALE
                                + pe_v[c * CHUNK + jj, sl]
                            )
                    return c2

                lax.fori_loop(0, CHUNK // 4, rbody, 0)
                pltpu.async_copy(
                    rows_out[c],
                    out_hbm.at[seq, pl.ds(c * CHUNK, CHUNK)],
                    ssem[c],
                )

                @pl.when(i < seq_per_w - 1)
                def _fire_next(i=i, c=c):
                    u_next = NCHUNK * (i + 1) + c
                    u_next = u_next - jnp.where(
                        i >= half_i - 1, NCHUNK * half_i, 0
                    )
                    pltpu.async_copy(
                        table_hbm.at[idx_v.at[u_next]],
                        rows_in[c],
                        gsem[c],
                    )
            return carry

        lax.fori_loop(0, seq_per_w, body, 0)
        for c in range(NCHUNK):
            pltpu.make_async_copy(
                rows_out[c],
                out_hbm.at[sbase, pl.ds(c * CHUNK, CHUNK)],
                ssem[c],
            ).wait()

    return k


_sc_kernel = _make_sc_kernel()


def kernel(inputs, table):
    idx = inputs.reshape(UNITS, CHUNK)
    return _sc_kernel(idx, table, jnp.asarray(_PE))


# speedup vs baseline: 8.8927x; 1.3987x over previous
import functools

import jax
import jax.numpy as jnp
import numpy as np
from jax import lax
from jax.experimental import pallas as pl
from jax.experimental.pallas import tpu as pltpu
from jax.experimental.pallas import tpu_sc as plsc

VOCAB = 100000
D_MODEL = 128
BATCH = 4096
SEQ = 200
SCALE = float(np.sqrt(D_MODEL))

CHUNK = 40
NCHUNK = SEQ // CHUNK
UNITS = BATCH * NCHUNK


def _positional_encoding(length, depth):
    positions = np.arange(length)[:, np.newaxis]
    depths = np.arange(depth // 2)[np.newaxis, :] / (depth // 2)
    angle_rates = 1.0 / (10000 ** depths)
    angle_rads = positions * angle_rates
    return np.concatenate(
        [np.sin(angle_rads), np.cos(angle_rads)], axis=-1
    ).astype(np.float32)


_PE = _positional_encoding(SEQ, D_MODEL)


def _make_sc_kernel():
    info = plsc.get_sparse_core_info()
    nc, ns, lanes = info.num_cores, info.num_subcores, info.num_lanes
    nw = nc * ns
    seq_per_w = BATCH // nw
    u_per_w = UNITS // nw
    nvec = D_MODEL // lanes
    mesh = plsc.VectorSubcoreMesh(core_axis_name="c", subcore_axis_name="s")

    @functools.partial(
        pl.kernel,
        mesh=mesh,
        out_type=jax.ShapeDtypeStruct((BATCH, SEQ, D_MODEL), jnp.float32),
        scratch_types=[
            pltpu.VMEM((u_per_w // 2, CHUNK), jnp.int32),
            pltpu.VMEM((SEQ, D_MODEL), jnp.float32),
        ]
        + [pltpu.VMEM((CHUNK, D_MODEL), jnp.float32)] * (2 * NCHUNK)
        + [pltpu.SemaphoreType.DMA] * (2 * NCHUNK),
    )
    def k(idx_hbm, table_hbm, pe_hbm, out_hbm, idx_v, pe_v, *bufs):
        rows_in = bufs[:NCHUNK]
        rows_out = bufs[NCHUNK:2 * NCHUNK]
        gsem = bufs[2 * NCHUNK:3 * NCHUNK]
        ssem = bufs[3 * NCHUNK:4 * NCHUNK]
        wid = lax.axis_index("s") * nc + lax.axis_index("c")
        sbase = wid * seq_per_w
        half_u = u_per_w // 2
        half_i = seq_per_w // 2
        pltpu.sync_copy(pe_hbm, pe_v)
        pltpu.sync_copy(idx_hbm.at[pl.ds(wid * u_per_w, half_u)], idx_v)
        for c in range(NCHUNK):
            pltpu.async_copy(table_hbm.at[idx_v.at[c]], rows_in[c], gsem[c])

        def body(i, carry):
            seq = sbase + i

            @pl.when(i == half_i - 1)
            def _refill_idx():
                for c in range(NCHUNK):
                    pltpu.make_async_copy(
                        out_hbm.at[sbase, pl.ds(c * CHUNK, CHUNK)],
                        rows_in[c], gsem[c],
                    ).wait()
                pltpu.sync_copy(
                    idx_hbm.at[pl.ds(wid * u_per_w + half_u, half_u)], idx_v
                )

            for c in range(NCHUNK):
                @pl.when(i != half_i - 1)
                def _wait_gather(c=c):
                    pltpu.make_async_copy(
                        out_hbm.at[sbase, pl.ds(c * CHUNK, CHUNK)],
                        rows_in[c], gsem[c],
                    ).wait()

                @pl.when(i > 0)
                def _wait_store(c=c):
                    pltpu.make_async_copy(
                        rows_out[c],
                        out_hbm.at[sbase, pl.ds(c * CHUNK, CHUNK)],
                        ssem[c],
                    ).wait()

                def rbody(j, c2, c=c):
                    for r in range(4):
                        jj = 4 * j + r
                        for v in range(nvec):
                            sl = pl.ds(v * lanes, lanes)
                            rows_out[c][jj, sl] = (
                                rows_in[c][jj, sl] * SCALE
                                + pe_v[c * CHUNK + jj, sl]
                            )
                    return c2

                lax.fori_loop(0, CHUNK // 4, rbody, 0)
                pltpu.async_copy(
                    rows_out[c],
                    out_hbm.at[seq, pl.ds(c * CHUNK, CHUNK)],
                    ssem[c],
                )

                @pl.when(i < seq_per_w - 1)
                def _fire_next(i=i, c=c):
                    u_next = NCHUNK * (i + 1) + c
                    u_next = u_next - jnp.where(
                        i >= half_i - 1, NCHUNK * half_i, 0
                    )
                    pltpu.async_copy(
                        table_hbm.at[idx_v.at[u_next]],
                        rows_in[c],
                        gsem[c],
                    )
            return carry

        lax.fori_loop(0, seq_per_w, body, 0)
        for c in range(NCHUNK):
            pltpu.make_async_copy(
                rows_out[c],
                out_hbm.at[sbase, pl.ds(c * CHUNK, CHUNK)],
                ssem[c],
            ).wait()

    return k


_sc_kernel = _make_sc_kernel()


def kernel(inputs, table):
    idx = inputs.reshape(UNITS, CHUNK)
    return _sc_kernel(idx, table, jnp.asarray(_PE))
